# baseline (speedup 1.0000x reference)
import jax
import jax.numpy as jnp
from jax.experimental import pallas as pl
from jax.experimental.pallas import tpu as pltpu

B = 32
S = 1
HIDDEN = 2048
DFF = 768
E = 64
TOPK = 8
T = B * S
FH = DFF // 2
HH = HIDDEN // 2
NSTEP = 2 * E


def _moe_kernel(x_ref, gate_w_ref, wg_ref, wu_ref, wd_ref, out_ref,
                rw_ref, acc0_ref, acc1_ref, glu0_ref, glu1_ref):
    s = pl.program_id(0)
    even = (s % 2) == 0

    @pl.when(s == 0)
    def _router():
        x = x_ref[...]
        logits = jax.lax.dot_general(
            x, gate_w_ref[...],
            (((1,), (1,)), ((), ())),
            preferred_element_type=jnp.float32)
        col = jax.lax.broadcasted_iota(jnp.int32, (T, E), 1)
        neg_inf = jnp.float32(-jnp.inf)
        cur = logits
        sel = jnp.zeros((T, E), dtype=jnp.bool_)
        for _ in range(TOPK):
            mx = jnp.max(cur, axis=1, keepdims=True)
            at_max = cur == mx
            first = jnp.min(jnp.where(at_max, col, E), axis=1, keepdims=True)
            pick = col == first
            sel = jnp.logical_or(sel, pick)
            cur = jnp.where(pick, neg_inf, cur)
        z = jnp.where(sel, logits, neg_inf)
        zmax = jnp.max(z, axis=1, keepdims=True)
        p = jnp.where(sel, jnp.exp(z - zmax), 0.0)
        rw_ref[...] = p / jnp.sum(p, axis=1, keepdims=True)
        acc0_ref[...] = jnp.zeros_like(acc0_ref)
        acc1_ref[...] = jnp.zeros_like(acc1_ref)

    def _down(acc_ref):
        e_d = (s - 1) // 2
        wd = wd_ref[0, 0]
        o = (jax.lax.dot_general(glu0_ref[...], wd[:, :FH],
                                 (((1,), (1,)), ((), ())),
                                 preferred_element_type=jnp.float32) +
             jax.lax.dot_general(glu1_ref[...], wd[:, FH:],
                                 (((1,), (1,)), ((), ())),
                                 preferred_element_type=jnp.float32))
        rw = rw_ref[...]
        ecol = jax.lax.broadcasted_iota(jnp.int32, (T, E), 1)
        w_col = jnp.sum(jnp.where(ecol == e_d, rw, 0.0),
                        axis=1, keepdims=True)
        acc_ref[...] += w_col * o

    def _gate_up(glu_ref):
        x = x_ref[...]
        g = jax.lax.dot_general(x, wg_ref[0, 0], (((1,), (1,)), ((), ())),
                                preferred_element_type=jnp.float32)
        u = jax.lax.dot_general(x, wu_ref[0, 0], (((1,), (1,)), ((), ())),
                                preferred_element_type=jnp.float32)
        glu_ref[...] = g * jax.nn.sigmoid(g) * u

    @pl.when(jnp.logical_and(s > 0, even))
    def _down_even():
        _down(acc1_ref)

    @pl.when(jnp.logical_and(s < NSTEP, even))
    def _gu_even():
        _gate_up(glu0_ref)

    @pl.when(jnp.logical_not(even))
    def _gu_odd():
        _gate_up(glu1_ref)
        _down(acc0_ref)

    @pl.when(s == NSTEP)
    def _write():
        out_ref[:, :HH] = acc0_ref[...]
        out_ref[:, HH:] = acc1_ref[...]


def kernel(hidden_states, gate_w, w_gate, w_up, w_down):
    x = hidden_states.reshape(T, HIDDEN)
    wg = w_gate.reshape(E, 2, FH, HIDDEN)
    wu = w_up.reshape(E, 2, FH, HIDDEN)
    wd = w_down.reshape(E, 2, HH, DFF)
    lastf = NSTEP - 1
    out = pl.pallas_call(
        _moe_kernel,
        grid=(NSTEP + 1,),
        in_specs=[
            pl.BlockSpec((T, HIDDEN), lambda s: (0, 0)),
            pl.BlockSpec((E, HIDDEN), lambda s: (0, 0)),
            pl.BlockSpec((1, 1, FH, HIDDEN),
                         lambda s: (jnp.minimum(s, lastf) // 2,
                                    jnp.minimum(s, lastf) % 2, 0, 0)),
            pl.BlockSpec((1, 1, FH, HIDDEN),
                         lambda s: (jnp.minimum(s, lastf) // 2,
                                    jnp.minimum(s, lastf) % 2, 0, 0)),
            pl.BlockSpec((1, 1, HH, DFF),
                         lambda s: (jnp.maximum(s - 1, 0) // 2,
                                    jnp.maximum(s - 1, 0) % 2, 0, 0)),
        ],
        out_specs=pl.BlockSpec((T, HIDDEN), lambda s: (0, 0)),
        out_shape=jax.ShapeDtypeStruct((T, HIDDEN), jnp.float32),
        scratch_shapes=[
            pltpu.VMEM((T, E), jnp.float32),
            pltpu.VMEM((T, HH), jnp.float32),
            pltpu.VMEM((T, HH), jnp.float32),
            pltpu.VMEM((T, FH), jnp.float32),
            pltpu.VMEM((T, FH), jnp.float32),
        ],
    )(x, gate_w, wg, wu, wd)
    return out.reshape(B, S, HIDDEN)
